# Initial kernel scaffold; baseline (speedup 1.0000x reference)
import jax
import jax.numpy as jnp
from jax.experimental import pallas as pl


def kernel(x, edge_index, edge_weight, h, c, Wx, bx, Wh, bh, w_peep, b_gate):
    raise NotImplementedError("write your pallas kernel here")


# speedup vs baseline: 11.0097x; 11.0097x over previous
import functools

import jax
import jax.numpy as jnp
from jax import lax
from jax.experimental import pallas as pl
from jax.experimental.pallas import tpu as pltpu
from jax.experimental.pallas import tpu_sc as plsc

N = 10000
E = 320000
HID = 128
NC = 2
NS = 16
NPAD = 10240
NPS = NPAD // NS
EPT = E // NS
C = 80
NCH = EPT // C
L = 16


def _rsqrt_newton(d):
    i = plsc.bitcast(d, jnp.int32)
    i = jnp.int32(0x5F3759DF) - (i >> 1)
    y = plsc.bitcast(i, jnp.float32)
    for _ in range(4):
        y = y * (1.5 - 0.5 * d * y * y)
    return jnp.where(d > 0, y, 0.0)


G = 25
NG = NCH // G


def _sc_body(vsplit, src3, dst3, w3, out, degparts, dinvall, srcg, dstg, wg,
             nodebuf, redb, rowb, dloc, srcadj, normv, accsh, sem):
    ci = lax.axis_index("c")
    si = lax.axis_index("s")

    def _zero_deg(k, _):
        nodebuf[pl.ds(k * L, L)] = jnp.zeros((L,), jnp.float32)
        return 0
    lax.fori_loop(0, NPAD // L, _zero_deg, 0)

    def _deg_group(g, _):
        pltpu.sync_copy(src3.at[si, g], srcg)
        pltpu.sync_copy(w3.at[si, g], wg)

        def _deg_chunk(ch, _):
            for j in range(C // L):
                sidx = srcg[ch, pl.ds(j * L, L)]
                wval = wg[ch, pl.ds(j * L, L)]
                plsc.addupdate_scatter(nodebuf, [sidx], wval)
            return 0
        lax.fori_loop(0, G, _deg_chunk, 0)
        return 0
    lax.fori_loop(0, NG, _deg_group, 0)

    pltpu.sync_copy(nodebuf, degparts.at[ci, si])
    plsc.subcore_barrier()

    for p in range(NS):
        pltpu.sync_copy(degparts.at[ci, p, pl.ds(si * NPS, NPS)], redb.at[p])

    def _dinv_vec(j, _):
        d = redb[0, pl.ds(j * L, L)]
        for p in range(1, NS):
            d = d + redb[p, pl.ds(j * L, L)]
        dloc[pl.ds(j * L, L)] = _rsqrt_newton(d)
        return 0
    lax.fori_loop(0, NPS // L, _dinv_vec, 0)

    pltpu.sync_copy(dloc, dinvall.at[ci, pl.ds(si * NPS, NPS)])
    plsc.subcore_barrier()
    pltpu.sync_copy(dinvall.at[ci], nodebuf)

    def _zero_row(r, _):
        for j in range(HID // L):
            rowb[r, pl.ds(j * L, L)] = jnp.zeros((L,), jnp.float32)
        return 0
    lax.fori_loop(0, C, _zero_row, 0)
    for t in range(NPS // C):
        pltpu.sync_copy(rowb, accsh.at[pl.ds(si * NPS + t * C, C), :])
    plsc.subcore_barrier()

    coff = ci * N

    def _group(g, _):
        pltpu.sync_copy(src3.at[si, g], srcg)
        pltpu.sync_copy(dst3.at[si, g], dstg)
        pltpu.sync_copy(w3.at[si, g], wg)

        def _chunk(ch, _):
            for j in range(C // L):
                sidx = srcg[ch, pl.ds(j * L, L)]
                didx = dstg[ch, pl.ds(j * L, L)]
                wval = wg[ch, pl.ds(j * L, L)]
                nsrc = plsc.load_gather(nodebuf, [sidx])
                ndst = plsc.load_gather(nodebuf, [didx])
                normv[pl.ds(j * L, L)] = -(nsrc * wval * ndst)
                srcadj[pl.ds(j * L, L)] = sidx + coff
            pltpu.async_copy(vsplit.at[srcadj], rowb, sem).wait()

            def _scale(r, _):
                nb = plsc.load_gather(normv, [jnp.full((L,), r, jnp.int32)])
                for j in range(HID // L):
                    rowb[r, pl.ds(j * L, L)] = rowb[r, pl.ds(j * L, L)] * nb
                return 0
            lax.fori_loop(0, C, _scale, 0)

            pltpu.sync_copy(rowb, accsh.at[dstg.at[ch]], add=True)
            return 0
        lax.fori_loop(0, G, _chunk, 0)
        return 0
    lax.fori_loop(0, NG, _group, 0)

    plsc.subcore_barrier()
    pltpu.sync_copy(accsh.at[pl.ds(si * NPS, NPS), :],
                    out.at[ci, pl.ds(si * NPS, NPS), :])


def _sc_aggregate(x, h, src, dst, w):
    vsplit = jnp.concatenate([x, h], axis=0)
    src3 = src.reshape(NS, NG, G, C)
    dst3 = dst.reshape(NS, NG, G, C)
    w3 = w.reshape(NS, NG, G, C)
    mesh = plsc.VectorSubcoreMesh(core_axis_name="c", subcore_axis_name="s",
                                  num_cores=NC, num_subcores=NS)
    f = pl.kernel(
        _sc_body,
        out_type=[
            jax.ShapeDtypeStruct((NC, NPAD, HID), jnp.float32),
            jax.ShapeDtypeStruct((NC, NS, NPAD), jnp.float32),
            jax.ShapeDtypeStruct((NC, NPAD), jnp.float32),
        ],
        mesh=mesh,
        scratch_types=[
            pltpu.VMEM((G, C), jnp.int32),
            pltpu.VMEM((G, C), jnp.int32),
            pltpu.VMEM((G, C), jnp.float32),
            pltpu.VMEM((NPAD,), jnp.float32),
            pltpu.VMEM((NS, NPS), jnp.float32),
            pltpu.VMEM((C, HID), jnp.float32),
            pltpu.VMEM((NPS,), jnp.float32),
            pltpu.VMEM((C,), jnp.int32),
            pltpu.VMEM((C,), jnp.float32),
            pltpu.VMEM_SHARED((NPAD, HID), jnp.float32),
            pltpu.SemaphoreType.DMA,
        ],
        compiler_params=pltpu.CompilerParams(needs_layout_passes=False),
    )
    S, _, _ = f(vsplit, src3, dst3, w3)
    return S


R = 2000


def _tc_body(u_ref, w_ref, b_ref, wp_ref, c_ref, hn_ref, cn_ref):
    g = jnp.dot(u_ref[...], w_ref[...],
                preferred_element_type=jnp.float32) + b_ref[...]
    cc = c_ref[...]
    gi = jax.nn.sigmoid(g[:, 0:HID] + wp_ref[0:1, :] * cc)
    gf = jax.nn.sigmoid(g[:, HID:2 * HID] + wp_ref[1:2, :] * cc)
    gt = jnp.tanh(g[:, 2 * HID:3 * HID])
    cn = gf * cc + gi * gt
    go = jax.nn.sigmoid(g[:, 3 * HID:4 * HID] + wp_ref[2:3, :] * cn)
    hn_ref[...] = go * jnp.tanh(cn)
    cn_ref[...] = cn


def _tc_gates(U, Wbig, bias, w_peep, c):
    hn, cn = pl.pallas_call(
        _tc_body,
        grid=(N // R,),
        in_specs=[
            pl.BlockSpec((R, 4 * HID), lambda i: (i, 0)),
            pl.BlockSpec((4 * HID, 4 * HID), lambda i: (0, 0)),
            pl.BlockSpec((1, 4 * HID), lambda i: (0, 0)),
            pl.BlockSpec((3, HID), lambda i: (0, 0)),
            pl.BlockSpec((R, HID), lambda i: (i, 0)),
        ],
        out_specs=[
            pl.BlockSpec((R, HID), lambda i: (i, 0)),
            pl.BlockSpec((R, HID), lambda i: (i, 0)),
        ],
        out_shape=[
            jax.ShapeDtypeStruct((N, HID), jnp.float32),
            jax.ShapeDtypeStruct((N, HID), jnp.float32),
        ],
    )(U, Wbig, bias, w_peep, c)
    return hn, cn


def kernel(x, edge_index, edge_weight, h, c, Wx, bx, Wh, bh, w_peep, b_gate):
    src = edge_index[0]
    dst = edge_index[1]

    S = _sc_aggregate(x, h, src, dst, edge_weight)
    Sx = S[0, :N, :]
    Sh = S[1, :N, :]

    U = jnp.concatenate([x, h, Sx, Sh], axis=1)
    Wbig = jnp.concatenate([Wx[:, 0], Wh[:, 0], Wx[:, 1], Wh[:, 1]],
                           axis=1)
    Wbig = jnp.transpose(Wbig, (1, 0, 2)).reshape(4 * HID, 4 * HID)
    bias = (bx + bh + b_gate).reshape(1, 4 * HID)

    hn, cn = _tc_gates(U, Wbig, bias, w_peep, c)
    return (hn, hn, cn)


# speedup vs baseline: 16.4999x; 1.4987x over previous
import functools

import jax
import jax.numpy as jnp
from jax import lax
from jax.experimental import pallas as pl
from jax.experimental.pallas import tpu as pltpu
from jax.experimental.pallas import tpu_sc as plsc

N = 10000
E = 320000
HID = 128
NC = 2
NS = 16
NPAD = 10240
NPS = NPAD // NS
EPT = E // NS
C = 80
NCH = EPT // C
L = 16


def _rsqrt_newton(d):
    i = plsc.bitcast(d, jnp.int32)
    i = jnp.int32(0x5F3759DF) - (i >> 1)
    y = plsc.bitcast(i, jnp.float32)
    for _ in range(4):
        y = y * (1.5 - 0.5 * d * y * y)
    return jnp.where(d > 0, y, 0.0)


G = 25
NG = NCH // G


def _sc_body(vsplit, src3, dst3, w3, out, degparts, dinvall, srcg, dstg, wg,
             nodebuf, redb, rowb, rowb2, dloc, sadjA, sadjB, normA, normB,
             accsh, sem, gsemA, gsemB, ssem):
    ci = lax.axis_index("c")
    si = lax.axis_index("s")

    def _zero_deg(k, _):
        nodebuf[pl.ds(k * L, L)] = jnp.zeros((L,), jnp.float32)
        return 0
    lax.fori_loop(0, NPAD // L, _zero_deg, 0)

    def _deg_group(g, _):
        pltpu.sync_copy(src3.at[si, g], srcg)
        pltpu.sync_copy(w3.at[si, g], wg)

        def _deg_chunk(ch, _):
            for j in range(C // L):
                sidx = srcg[ch, pl.ds(j * L, L)]
                wval = wg[ch, pl.ds(j * L, L)]
                plsc.addupdate_scatter(nodebuf, [sidx], wval)
            return 0
        lax.fori_loop(0, G, _deg_chunk, 0)
        return 0
    lax.fori_loop(0, NG, _deg_group, 0)

    pltpu.sync_copy(nodebuf, degparts.at[ci, si])
    plsc.subcore_barrier()

    STR = 128
    for t in range(NPS // STR):
        for p in range(NS):
            pltpu.sync_copy(
                degparts.at[ci, p, pl.ds(si * NPS + t * STR, STR)], redb.at[p])

        def _dinv_vec(j, _):
            d = redb[0, pl.ds(j * L, L)]
            for p in range(1, NS):
                d = d + redb[p, pl.ds(j * L, L)]
            dloc[pl.ds(t * STR + j * L, L)] = _rsqrt_newton(d)
            return 0
        lax.fori_loop(0, STR // L, _dinv_vec, 0)

    pltpu.sync_copy(dloc, dinvall.at[ci, pl.ds(si * NPS, NPS)])
    plsc.subcore_barrier()
    pltpu.sync_copy(dinvall.at[ci], nodebuf)

    def _zero_row(r, _):
        for j in range(HID // L):
            rowb[r, pl.ds(j * L, L)] = jnp.zeros((L,), jnp.float32)
        return 0
    lax.fori_loop(0, C, _zero_row, 0)
    for t in range(NPS // C):
        pltpu.sync_copy(rowb, accsh.at[pl.ds(si * NPS + t * C, C), :])
    plsc.subcore_barrier()

    coff = ci * N

    def _prep(ch, sadj, normb):
        for j in range(C // L):
            sidx = srcg[ch, pl.ds(j * L, L)]
            didx = dstg[ch, pl.ds(j * L, L)]
            wval = wg[ch, pl.ds(j * L, L)]
            nsrc = plsc.load_gather(nodebuf, [sidx])
            ndst = plsc.load_gather(nodebuf, [didx])
            normb[pl.ds(j * L, L)] = -(nsrc * wval * ndst)
            sadj[pl.ds(j * L, L)] = sidx + coff

    def _gather(rb, sadj, gsem):
        pltpu.async_copy(vsplit.at[sadj], rb, gsem)

    def _drain_g(gsem):
        pltpu.make_async_copy(vsplit.at[pl.ds(0, C)], rowb, gsem).wait()

    def _scale(rb, normb):
        def _row(r, _):
            nb = plsc.load_gather(normb, [jnp.full((L,), r, jnp.int32)])
            for j in range(HID // L):
                rb[r, pl.ds(j * L, L)] = rb[r, pl.ds(j * L, L)] * nb
            return 0
        lax.fori_loop(0, C, _row, 0)

    def _scatter(rb, ch):
        pltpu.async_copy(rb, accsh.at[dstg.at[ch]], ssem, add=True)

    def _drain_s():
        pltpu.make_async_copy(vsplit.at[pl.ds(0, C)], rowb, ssem).wait()

    def _group(g, _):
        c1 = pltpu.async_copy(src3.at[si, g], srcg, sem)
        c2 = pltpu.async_copy(dst3.at[si, g], dstg, sem)
        c3 = pltpu.async_copy(w3.at[si, g], wg, sem)
        c1.wait(); c2.wait(); c3.wait()

        _prep(0, sadjA, normA)
        _gather(rowb, sadjA, gsemA)
        _prep(1, sadjB, normB)
        _gather(rowb2, sadjB, gsemB)

        def _pair(p, _):
            c0 = 2 * p
            _drain_g(gsemA)
            _scale(rowb, normA)
            _scatter(rowb, c0)
            _prep(c0 + 2, sadjA, normA)
            _drain_s()
            _gather(rowb, sadjA, gsemA)

            _drain_g(gsemB)
            _scale(rowb2, normB)
            _scatter(rowb2, c0 + 1)

            @pl.when(p < (G - 3) // 2)
            def _():
                _prep(c0 + 3, sadjB, normB)
                _drain_s()
                _gather(rowb2, sadjB, gsemB)
            return 0
        lax.fori_loop(0, (G - 1) // 2, _pair, 0)

        _drain_g(gsemA)
        _scale(rowb, normA)
        _scatter(rowb, G - 1)
        _drain_s()
        _drain_s()
        return 0
    lax.fori_loop(0, NG, _group, 0)

    plsc.subcore_barrier()
    pltpu.sync_copy(accsh.at[pl.ds(si * NPS, NPS), :],
                    out.at[ci, pl.ds(si * NPS, NPS), :])


def _sc_aggregate(x, h, src, dst, w):
    vsplit = jnp.concatenate([x, h], axis=0)
    src3 = src.reshape(NS, NG, G, C)
    dst3 = dst.reshape(NS, NG, G, C)
    w3 = w.reshape(NS, NG, G, C)
    mesh = plsc.VectorSubcoreMesh(core_axis_name="c", subcore_axis_name="s",
                                  num_cores=NC, num_subcores=NS)
    f = pl.kernel(
        _sc_body,
        out_type=[
            jax.ShapeDtypeStruct((NC, NPAD, HID), jnp.float32),
            jax.ShapeDtypeStruct((NC, NS, NPAD), jnp.float32),
            jax.ShapeDtypeStruct((NC, NPAD), jnp.float32),
        ],
        mesh=mesh,
        scratch_types=[
            pltpu.VMEM((G, C), jnp.int32),
            pltpu.VMEM((G, C), jnp.int32),
            pltpu.VMEM((G, C), jnp.float32),
            pltpu.VMEM((NPAD,), jnp.float32),
            pltpu.VMEM((NS, 128), jnp.float32),
            pltpu.VMEM((C, HID), jnp.float32),
            pltpu.VMEM((C, HID), jnp.float32),
            pltpu.VMEM((NPS,), jnp.float32),
            pltpu.VMEM((C,), jnp.int32),
            pltpu.VMEM((C,), jnp.int32),
            pltpu.VMEM((C,), jnp.float32),
            pltpu.VMEM((C,), jnp.float32),
            pltpu.VMEM_SHARED((NPAD, HID), jnp.float32),
            pltpu.SemaphoreType.DMA,
            pltpu.SemaphoreType.DMA,
            pltpu.SemaphoreType.DMA,
            pltpu.SemaphoreType.DMA,
        ],
        compiler_params=pltpu.CompilerParams(needs_layout_passes=False),
    )
    S, _, _ = f(vsplit, src3, dst3, w3)
    return S


R = 2000


def _tc_body(u_ref, w_ref, b_ref, wp_ref, c_ref, hn_ref, cn_ref):
    g = jnp.dot(u_ref[...], w_ref[...],
                preferred_element_type=jnp.float32) + b_ref[...]
    cc = c_ref[...]
    gi = jax.nn.sigmoid(g[:, 0:HID] + wp_ref[0:1, :] * cc)
    gf = jax.nn.sigmoid(g[:, HID:2 * HID] + wp_ref[1:2, :] * cc)
    gt = jnp.tanh(g[:, 2 * HID:3 * HID])
    cn = gf * cc + gi * gt
    go = jax.nn.sigmoid(g[:, 3 * HID:4 * HID] + wp_ref[2:3, :] * cn)
    hn_ref[...] = go * jnp.tanh(cn)
    cn_ref[...] = cn


def _tc_gates(U, Wbig, bias, w_peep, c):
    hn, cn = pl.pallas_call(
        _tc_body,
        grid=(N // R,),
        in_specs=[
            pl.BlockSpec((R, 4 * HID), lambda i: (i, 0)),
            pl.BlockSpec((4 * HID, 4 * HID), lambda i: (0, 0)),
            pl.BlockSpec((1, 4 * HID), lambda i: (0, 0)),
            pl.BlockSpec((3, HID), lambda i: (0, 0)),
            pl.BlockSpec((R, HID), lambda i: (i, 0)),
        ],
        out_specs=[
            pl.BlockSpec((R, HID), lambda i: (i, 0)),
            pl.BlockSpec((R, HID), lambda i: (i, 0)),
        ],
        out_shape=[
            jax.ShapeDtypeStruct((N, HID), jnp.float32),
            jax.ShapeDtypeStruct((N, HID), jnp.float32),
        ],
    )(U, Wbig, bias, w_peep, c)
    return hn, cn


def kernel(x, edge_index, edge_weight, h, c, Wx, bx, Wh, bh, w_peep, b_gate):
    src = edge_index[0]
    dst = edge_index[1]

    S = _sc_aggregate(x, h, src, dst, edge_weight)
    Sx = S[0, :N, :]
    Sh = S[1, :N, :]

    U = jnp.concatenate([x, h, Sx, Sh], axis=1)
    Wbig = jnp.concatenate([Wx[:, 0], Wh[:, 0], Wx[:, 1], Wh[:, 1]],
                           axis=1)
    Wbig = jnp.transpose(Wbig, (1, 0, 2)).reshape(4 * HID, 4 * HID)
    bias = (bx + bh + b_gate).reshape(1, 4 * HID)

    hn, cn = _tc_gates(U, Wbig, bias, w_peep, c)
    return (hn, hn, cn)


# speedup vs baseline: 22.3503x; 1.3546x over previous
import functools

import jax
import jax.numpy as jnp
from jax import lax
from jax.experimental import pallas as pl
from jax.experimental.pallas import tpu as pltpu
from jax.experimental.pallas import tpu_sc as plsc

N = 10000
E = 320000
HID = 128
NC = 2
NS = 16
NPAD = 10240
NPS = NPAD // NS
EPT = E // NS
C = 80
NCH = EPT // C
L = 16


def _rsqrt_newton(d):
    i = plsc.bitcast(d, jnp.int32)
    i = jnp.int32(0x5F3759DF) - (i >> 1)
    y = plsc.bitcast(i, jnp.float32)
    for _ in range(4):
        y = y * (1.5 - 0.5 * d * y * y)
    return jnp.where(d > 0, y, 0.0)


G = 25
NG = NCH // G


def _sc_body(vsplit, e4, w4, out, degparts, dinvall, srcg, dstg, wg,
             nodebuf, redb, rowb, rowb2, dloc, sadjA, sadjB, normA, normB,
             accsh, sem, gsemA, gsemB, ssem):
    ci = lax.axis_index("c")
    si = lax.axis_index("s")

    def _zero_deg(k, _):
        nodebuf[pl.ds(k * L, L)] = jnp.zeros((L,), jnp.float32)
        return 0
    lax.fori_loop(0, NPAD // L, _zero_deg, 0)

    def _deg_group(g, _):
        d1 = pltpu.async_copy(e4.at[0, si, g], srcg, sem)
        d2 = pltpu.async_copy(w4.at[si, g], wg, sem)
        d1.wait(); d2.wait()

        def _deg_chunk(ch, _):
            for j in range(C // L):
                sidx = srcg[ch, pl.ds(j * L, L)]
                wval = wg[ch, pl.ds(j * L, L)]
                plsc.addupdate_scatter(nodebuf, [sidx], wval)
            return 0
        lax.fori_loop(0, G, _deg_chunk, 0)
        return 0
    lax.fori_loop(0, NG, _deg_group, 0)

    pltpu.sync_copy(nodebuf, degparts.at[ci, si])
    plsc.subcore_barrier()

    STR = 128
    for t in range(NPS // STR):
        pltpu.sync_copy(
            degparts.at[ci, :, pl.ds(si * NPS + t * STR, STR)], redb)

        def _dinv_vec(j, _):
            d = redb[0, pl.ds(j * L, L)]
            for p in range(1, NS):
                d = d + redb[p, pl.ds(j * L, L)]
            dloc[pl.ds(t * STR + j * L, L)] = _rsqrt_newton(d)
            return 0
        lax.fori_loop(0, STR // L, _dinv_vec, 0)

    pltpu.sync_copy(dloc, dinvall.at[ci, pl.ds(si * NPS, NPS)])
    plsc.subcore_barrier()
    pltpu.sync_copy(dinvall.at[ci], nodebuf)

    def _zero_row(r, _):
        for j in range(HID // L):
            rowb[r, pl.ds(j * L, L)] = jnp.zeros((L,), jnp.float32)
            rowb2[r, pl.ds(j * L, L)] = jnp.zeros((L,), jnp.float32)
        return 0
    lax.fori_loop(0, C, _zero_row, 0)
    zdescs = []
    for t in range(NPS // C):
        zb = rowb if t % 2 == 0 else rowb2
        zdescs.append(pltpu.async_copy(
            zb, accsh.at[pl.ds(si * NPS + t * C, C), :], sem))
    for d in zdescs:
        d.wait()
    plsc.subcore_barrier()

    coff = ci * N

    def _prep(ch, sadj, normb):
        for j in range(C // L):
            sidx = srcg[ch, pl.ds(j * L, L)]
            didx = dstg[ch, pl.ds(j * L, L)]
            wval = wg[ch, pl.ds(j * L, L)]
            nsrc = plsc.load_gather(nodebuf, [sidx])
            ndst = plsc.load_gather(nodebuf, [didx])
            normb[pl.ds(j * L, L)] = -(nsrc * wval * ndst)
            sadj[pl.ds(j * L, L)] = sidx + coff

    def _gather(rb, sadj, gsem):
        pltpu.async_copy(vsplit.at[sadj], rb, gsem)

    def _drain_g(gsem):
        pltpu.make_async_copy(vsplit.at[pl.ds(0, C)], rowb, gsem).wait()

    def _scale(rb, normb):
        @plsc.parallel_loop(0, C, unroll=4)
        def _row(r):
            nb = plsc.load_gather(normb, [jnp.full((L,), r, jnp.int32)])
            for j in range(HID // L):
                rb[r, pl.ds(j * L, L)] = rb[r, pl.ds(j * L, L)] * nb

    def _scatter(rb, ch):
        pltpu.async_copy(rb, accsh.at[dstg.at[ch]], ssem, add=True)

    def _drain_s():
        pltpu.make_async_copy(vsplit.at[pl.ds(0, C)], rowb, ssem).wait()

    def _group(g, _):
        c1 = pltpu.async_copy(e4.at[0, si, g], srcg, sem)
        c2 = pltpu.async_copy(e4.at[1, si, g], dstg, sem)
        c3 = pltpu.async_copy(w4.at[si, g], wg, sem)
        c1.wait(); c2.wait(); c3.wait()

        _prep(0, sadjA, normA)
        _gather(rowb, sadjA, gsemA)
        _prep(1, sadjB, normB)
        _gather(rowb2, sadjB, gsemB)

        def _pair(p, _):
            c0 = 2 * p
            _drain_g(gsemA)
            _scale(rowb, normA)
            _scatter(rowb, c0)
            _prep(c0 + 2, sadjA, normA)
            _drain_s()
            _gather(rowb, sadjA, gsemA)

            _drain_g(gsemB)
            _scale(rowb2, normB)
            _scatter(rowb2, c0 + 1)

            @pl.when(p < (G - 3) // 2)
            def _():
                _prep(c0 + 3, sadjB, normB)
                _drain_s()
                _gather(rowb2, sadjB, gsemB)
            return 0
        lax.fori_loop(0, (G - 1) // 2, _pair, 0)

        _drain_g(gsemA)
        _scale(rowb, normA)
        _scatter(rowb, G - 1)
        _drain_s()
        _drain_s()
        return 0
    lax.fori_loop(0, NG, _group, 0)

    plsc.subcore_barrier()
    pltpu.sync_copy(accsh.at[pl.ds(si * NPS, NPS), :],
                    out.at[ci, pl.ds(si * NPS, NPS), :])


def _sc_aggregate(x, h, edge_index, w):
    vsplit = jnp.concatenate([x, h], axis=0)
    e4 = edge_index.reshape(2, NS, NG, G, C)
    w4 = w.reshape(NS, NG, G, C)
    mesh = plsc.VectorSubcoreMesh(core_axis_name="c", subcore_axis_name="s",
                                  num_cores=NC, num_subcores=NS)
    f = pl.kernel(
        _sc_body,
        out_type=[
            jax.ShapeDtypeStruct((NC, NPAD, HID), jnp.float32),
            jax.ShapeDtypeStruct((NC, NS, NPAD), jnp.float32),
            jax.ShapeDtypeStruct((NC, NPAD), jnp.float32),
        ],
        mesh=mesh,
        scratch_types=[
            pltpu.VMEM((G, C), jnp.int32),
            pltpu.VMEM((G, C), jnp.int32),
            pltpu.VMEM((G, C), jnp.float32),
            pltpu.VMEM((NPAD,), jnp.float32),
            pltpu.VMEM((NS, 128), jnp.float32),
            pltpu.VMEM((C, HID), jnp.float32),
            pltpu.VMEM((C, HID), jnp.float32),
            pltpu.VMEM((NPS,), jnp.float32),
            pltpu.VMEM((C,), jnp.int32),
            pltpu.VMEM((C,), jnp.int32),
            pltpu.VMEM((C,), jnp.float32),
            pltpu.VMEM((C,), jnp.float32),
            pltpu.VMEM_SHARED((NPAD, HID), jnp.float32),
            pltpu.SemaphoreType.DMA,
            pltpu.SemaphoreType.DMA,
            pltpu.SemaphoreType.DMA,
            pltpu.SemaphoreType.DMA,
        ],
        compiler_params=pltpu.CompilerParams(needs_layout_passes=False),
    )
    S, _, _ = f(vsplit, e4, w4)
    return S


R = 2000


def _tc_body(x_ref, h_ref, s_ref, w_ref, b_ref, wp_ref, c_ref, hn_ref, cn_ref):
    W = w_ref[...]
    g = (jnp.dot(x_ref[...], W[0:HID], preferred_element_type=jnp.float32)
         + jnp.dot(h_ref[...], W[HID:2 * HID], preferred_element_type=jnp.float32)
         + jnp.dot(s_ref[0], W[2 * HID:3 * HID], preferred_element_type=jnp.float32)
         + jnp.dot(s_ref[1], W[3 * HID:4 * HID], preferred_element_type=jnp.float32)
         + b_ref[...])
    cc = c_ref[...]
    gi = jax.nn.sigmoid(g[:, 0:HID] + wp_ref[0:1, :] * cc)
    gf = jax.nn.sigmoid(g[:, HID:2 * HID] + wp_ref[1:2, :] * cc)
    gt = jnp.tanh(g[:, 2 * HID:3 * HID])
    cn = gf * cc + gi * gt
    go = jax.nn.sigmoid(g[:, 3 * HID:4 * HID] + wp_ref[2:3, :] * cn)
    hn_ref[...] = go * jnp.tanh(cn)
    cn_ref[...] = cn


def _tc_gates(x, h, S, Wbig, bias, w_peep, c):
    hn, cn = pl.pallas_call(
        _tc_body,
        grid=(N // R,),
        in_specs=[
            pl.BlockSpec((R, HID), lambda i: (i, 0)),
            pl.BlockSpec((R, HID), lambda i: (i, 0)),
            pl.BlockSpec((2, R, HID), lambda i: (0, i, 0)),
            pl.BlockSpec((4 * HID, 4 * HID), lambda i: (0, 0)),
            pl.BlockSpec((1, 4 * HID), lambda i: (0, 0)),
            pl.BlockSpec((3, HID), lambda i: (0, 0)),
            pl.BlockSpec((R, HID), lambda i: (i, 0)),
        ],
        out_specs=[
            pl.BlockSpec((R, HID), lambda i: (i, 0)),
            pl.BlockSpec((R, HID), lambda i: (i, 0)),
        ],
        out_shape=[
            jax.ShapeDtypeStruct((N, HID), jnp.float32),
            jax.ShapeDtypeStruct((N, HID), jnp.float32),
        ],
    )(x, h, S, Wbig, bias, w_peep, c)
    return hn, cn


def kernel(x, edge_index, edge_weight, h, c, Wx, bx, Wh, bh, w_peep, b_gate):
    S = _sc_aggregate(x, h, edge_index, edge_weight)

    Wbig = jnp.concatenate([Wx[:, 0], Wh[:, 0], Wx[:, 1], Wh[:, 1]],
                           axis=1)
    Wbig = jnp.transpose(Wbig, (1, 0, 2)).reshape(4 * HID, 4 * HID)
    bias = (bx + bh + b_gate).reshape(1, 4 * HID)

    hn, cn = _tc_gates(x, h, S, Wbig, bias, w_peep, c)
    return (hn, hn, cn)
